# baseline (speedup 1.0000x reference)
import functools

import jax
import jax.numpy as jnp
from jax import lax
from jax.experimental import pallas as pl
from jax.experimental.pallas import tpu as pltpu
from jax.experimental.pallas import tpu_sc as plsc

_NC, _NS = 2, 16
_NW = _NC * _NS
_SEQ = 8192
_DIM = 2048
_ROWS_PER_W = _SEQ // _NW
_CHUNK = 16
_NCHUNKS = _ROWS_PER_W // _CHUNK


def _sc_body(w_hbm, out_hbm, buf, sems):
    wid = lax.axis_index("s") * _NC + lax.axis_index("c")
    base = wid * _ROWS_PER_W

    def cp_in(c, slot):
        return pltpu.make_async_copy(
            w_hbm.at[pl.ds(base + c * _CHUNK, _CHUNK), :],
            buf.at[slot],
            sems.at[slot],
        )

    def cp_out(c, slot):
        return pltpu.make_async_copy(
            buf.at[slot],
            out_hbm.at[pl.ds(base + c * _CHUNK, _CHUNK), :],
            sems.at[2 + slot],
        )

    cp_in(0, 0).start()
    cp_in(1, 1).start()
    for c in range(_NCHUNKS):
        slot = c % 2
        cp_in(c, slot).wait()
        cp_out(c, slot).start()
        if c + 2 < _NCHUNKS:
            cp_out(c, slot).wait()
            cp_in(c + 2, slot).start()
    cp_out(_NCHUNKS - 2, (_NCHUNKS - 2) % 2).wait()
    cp_out(_NCHUNKS - 1, (_NCHUNKS - 1) % 2).wait()


def kernel(x, emb_weight):
    seq_len = x.shape[1]
    dim = emb_weight.shape[1]
    mesh = plsc.VectorSubcoreMesh(core_axis_name="c", subcore_axis_name="s")
    k = functools.partial(
        pl.kernel,
        mesh=mesh,
        out_type=jax.ShapeDtypeStruct((seq_len, dim), emb_weight.dtype),
        scratch_types=[
            pltpu.VMEM((2, _CHUNK, dim), emb_weight.dtype),
            pltpu.SemaphoreType.DMA((4,)),
        ],
    )(_sc_body)
    out = k(emb_weight[:seq_len])
    return out[None]
